# Initial kernel scaffold; baseline (speedup 1.0000x reference)
import jax
import jax.numpy as jnp
from jax.experimental import pallas as pl


def kernel(x, edge_index, emb_c0, emb_c1, emb_c2, gcn1_w, gcn1_b, gcn2_w, gcn2_b, w_ih, w_hh, b_ih, b_hh):
    raise NotImplementedError("write your pallas kernel here")


# speedup vs baseline: 1.0049x; 1.0049x over previous
import jax
import jax.numpy as jnp
from jax.experimental import pallas as pl
from jax.experimental.pallas import tpu as pltpu

N = 10000
W = 12
OUT = 128
NCAT = 3


def _gcn_conv(h_in, ei, w, b):
    h = h_in @ w
    src = ei[0]
    dst = ei[1]
    loop = jnp.arange(N, dtype=src.dtype)
    src = jnp.concatenate([src, loop])
    dst = jnp.concatenate([dst, loop])
    deg = jnp.zeros((N,), dtype=jnp.float32).at[dst].add(1.0)
    dinv = jnp.where(deg > 0, 1.0 / jnp.sqrt(deg), 0.0)
    norm = dinv[src] * dinv[dst]
    msg = h[src] * norm[:, None]
    out = jax.ops.segment_sum(msg, dst, num_segments=N)
    return out + b


def _gru_kernel(seq_ref, wih_ref, whh_ref, bih_ref, bhh_ref, out_ref):
    B = seq_ref.shape[1]
    h = jnp.zeros((B, OUT), dtype=jnp.float32)
    wih = wih_ref[...]
    whh = whh_ref[...]
    bih = bih_ref[...]
    bhh = bhh_ref[...]
    for t in range(W):
        xt = seq_ref[t]
        gi = jnp.dot(xt, wih, preferred_element_type=jnp.float32) + bih
        gh = jnp.dot(h, whh, preferred_element_type=jnp.float32) + bhh
        i_r = gi[:, 0:OUT]
        i_z = gi[:, OUT:2 * OUT]
        i_n = gi[:, 2 * OUT:]
        h_r = gh[:, 0:OUT]
        h_z = gh[:, OUT:2 * OUT]
        h_n = gh[:, 2 * OUT:]
        r = jax.nn.sigmoid(i_r + h_r)
        z = jax.nn.sigmoid(i_z + h_z)
        n = jnp.tanh(i_n + r * h_n)
        h = (1.0 - z) * n + z * h
    out_ref[...] = h


def _gru_last_hidden_pallas(seq, w_ih, w_hh, b_ih, b_hh):
    BG = 400
    nblk = N // BG
    wihT = w_ih.T
    whhT = w_hh.T
    bih2 = b_ih.reshape(1, -1)
    bhh2 = b_hh.reshape(1, -1)
    return pl.pallas_call(
        _gru_kernel,
        grid=(nblk,),
        in_specs=[
            pl.BlockSpec((W, BG, OUT), lambda i: (0, i, 0)),
            pl.BlockSpec((OUT, 3 * OUT), lambda i: (0, 0)),
            pl.BlockSpec((OUT, 3 * OUT), lambda i: (0, 0)),
            pl.BlockSpec((1, 3 * OUT), lambda i: (0, 0)),
            pl.BlockSpec((1, 3 * OUT), lambda i: (0, 0)),
        ],
        out_specs=pl.BlockSpec((BG, OUT), lambda i: (i, 0)),
        out_shape=jax.ShapeDtypeStruct((N, OUT), jnp.float32),
    )(seq, wihT, whhT, bih2, bhh2)


def kernel(x, edge_index, emb_c0, emb_c1, emb_c2, gcn1_w, gcn1_b, gcn2_w, gcn2_b, w_ih, w_hh, b_ih, b_hh):
    embs = [emb_c0, emb_c1, emb_c2]
    outs = []
    for i in range(W):
        feat = x[i]
        cat_emb = jnp.concatenate(
            [embs[j][feat[:, j].astype(jnp.int32)] for j in range(NCAT)], axis=-1)
        new_x = jnp.concatenate([cat_emb, feat[:, NCAT:]], axis=-1)
        h = jax.nn.relu(_gcn_conv(new_x, edge_index[i], gcn1_w[i], gcn1_b[i]))
        h = _gcn_conv(h, edge_index[i], gcn2_w[i], gcn2_b[i])
        outs.append(h[None])
    seq = jnp.concatenate(outs, axis=0)
    return _gru_last_hidden_pallas(seq, w_ih, w_hh, b_ih, b_hh)


# speedup vs baseline: 11.5115x; 11.4554x over previous
import functools

import jax
import jax.numpy as jnp
from jax import lax
from jax.experimental import pallas as pl
from jax.experimental.pallas import tpu as pltpu
from jax.experimental.pallas import tpu_sc as plsc

N = 10000
W = 12
OUT = 128
NCAT = 3
NNUM = 16
CAT_DIM = 1000
E = 160000

NPAD = 10240
NC, NS = 2, 16
NWK = NC * NS
STRIPE = NPAD // NS

K1 = 79
ET1 = K1 * 128
K2 = 40
ET2 = K2 * 128
KE = 90
EROWS = NCAT * W * NPAD

BN = 2000
BG = 1000

_mesh = plsc.VectorSubcoreMesh(core_axis_name="c", subcore_axis_name="s")


@functools.partial(
    pl.kernel,
    out_type=jax.ShapeDtypeStruct((EROWS, OUT), jnp.float32),
    mesh=_mesh,
    scratch_types=[
        pltpu.VMEM((KE, 128), jnp.int32),
        pltpu.VMEM((128, OUT), jnp.float32),
        pltpu.SemaphoreType.DMA,
    ],
)
def _emb_gather(emb_hbm, eidx_hbm, out_hbm, idx_v, rows_v, sem):
    c = lax.axis_index("c")
    s = lax.axis_index("s")
    base = (c * NS + s) * (KE * 128)
    pltpu.sync_copy(eidx_hbm.at[c, s], idx_v)

    def body(j, carry):
        pltpu.async_copy(emb_hbm.at[idx_v.at[j]], rows_v, sem).wait()
        pltpu.sync_copy(rows_v, out_hbm.at[pl.ds(base + j * 128, 128)])
        return carry

    lax.fori_loop(0, KE, body, 0)


@functools.partial(
    pl.kernel,
    out_type=jax.ShapeDtypeStruct((W, NC, NPAD), jnp.float32),
    mesh=_mesh,
    scratch_types=[
        pltpu.VMEM((K2, 128), jnp.int32),
        pltpu.VMEM((128,), jnp.float32),
        pltpu.VMEM((STRIPE,), jnp.float32),
        pltpu.VMEM_SHARED((NPAD,), jnp.float32),
    ],
)
def _degree(didx_hbm, out_hbm, idx_v, ones_v, zeros_v, deg_sp):
    c = lax.axis_index("c")
    s = lax.axis_index("s")

    def fill(i, carry):
        ones_v[pl.ds(i * 16, 16)] = jnp.full((16,), 1.0, jnp.float32)
        return carry

    lax.fori_loop(0, 8, fill, 0)

    def zfill(i, carry):
        zeros_v[pl.ds(i * 16, 16)] = jnp.zeros((16,), jnp.float32)
        return carry

    lax.fori_loop(0, STRIPE // 16, zfill, 0)

    def win(w, carry):
        pltpu.sync_copy(zeros_v, deg_sp.at[pl.ds(s * STRIPE, STRIPE)])
        plsc.subcore_barrier()
        pltpu.sync_copy(didx_hbm.at[w, c, s], idx_v)

        def body(j, carry2):
            pltpu.sync_copy(ones_v, deg_sp.at[idx_v.at[j]], add=True)
            return carry2

        lax.fori_loop(0, K2, body, 0)
        plsc.subcore_barrier()
        pltpu.sync_copy(deg_sp.at[pl.ds(s * STRIPE, STRIPE)],
                        out_hbm.at[w, c, pl.ds(s * STRIPE, STRIPE)])
        plsc.subcore_barrier()
        return carry

    lax.fori_loop(0, W, win, 0)


def _make_conv(k_chunks):
    @functools.partial(
        pl.kernel,
        out_type=jax.ShapeDtypeStruct((W, NC, NPAD, OUT), jnp.float32),
        mesh=_mesh,
        scratch_types=[
            pltpu.VMEM((k_chunks, 128), jnp.int32),
            pltpu.VMEM((k_chunks, 128), jnp.int32),
            pltpu.VMEM((128, OUT), jnp.float32),
            pltpu.VMEM_SHARED((NPAD, OUT), jnp.float32),
            pltpu.SemaphoreType.DMA,
        ],
    )
    def conv(table_hbm, sidx_hbm, didx_hbm, zeros_hbm, out_hbm,
             sidx_v, didx_v, rows_v, acc_sp, sem):
        c = lax.axis_index("c")
        s = lax.axis_index("s")

        def win(w, carry):
            pltpu.sync_copy(zeros_hbm.at[pl.ds(s * STRIPE, STRIPE)],
                            acc_sp.at[pl.ds(s * STRIPE, STRIPE)])
            plsc.subcore_barrier()
            pltpu.sync_copy(sidx_hbm.at[w, c, s], sidx_v)
            pltpu.sync_copy(didx_hbm.at[w, c, s], didx_v)

            def body(j, carry2):
                pltpu.async_copy(table_hbm.at[sidx_v.at[j]], rows_v, sem).wait()
                pltpu.sync_copy(rows_v, acc_sp.at[didx_v.at[j]], add=True)
                return carry2

            lax.fori_loop(0, k_chunks, body, 0)
            plsc.subcore_barrier()
            pltpu.sync_copy(acc_sp.at[pl.ds(s * STRIPE, STRIPE)],
                            out_hbm.at[w, c, pl.ds(s * STRIPE, STRIPE)])
            plsc.subcore_barrier()
            return carry

        lax.fori_loop(0, W, win, 0)

    return conv


_conv1 = _make_conv(K1)
_conv2 = _make_conv(K2)


def _dense1_body(cat_ref, xnum_ref, deg_ref, w1c_ref, w1n_ref, h1s_ref, dinv_ref):
    deg = jnp.sum(deg_ref[0, :, :, 0], axis=0) + 1.0
    dinv = lax.rsqrt(deg)
    acc = jnp.dot(cat_ref[0, 0], w1c_ref[0, 0], preferred_element_type=jnp.float32)
    acc = acc + jnp.dot(cat_ref[1, 0], w1c_ref[0, 1], preferred_element_type=jnp.float32)
    acc = acc + jnp.dot(cat_ref[2, 0], w1c_ref[0, 2], preferred_element_type=jnp.float32)
    acc = acc + jnp.dot(xnum_ref[0], w1n_ref[0], preferred_element_type=jnp.float32)
    h1s = acc * dinv[:, None]
    h1s_ref[0, 0] = h1s[:, :OUT]
    h1s_ref[1, 0] = h1s[:, OUT:]
    dinv_ref[0, :, 0] = dinv


def _dense2_body(acc1_ref, h1s_ref, dinv_ref, b1_ref, w2_ref, h2_ref):
    dinv = dinv_ref[0, :, 0]
    res = None
    for half in range(2):
        pre = acc1_ref[0, half] + h1s_ref[half, 0]
        h1h = jnp.maximum(dinv[:, None] * pre + b1_ref[0, half], 0.0)
        d = jnp.dot(h1h, w2_ref[0, half], preferred_element_type=jnp.float32)
        res = d if res is None else res + d
    h2_ref[0] = res * dinv[:, None]


def _gru_body(acc2_ref, h2_ref, dinv_ref, b2_ref, wih_ref, whh_ref,
              bih_ref, bhh_ref, out_ref):
    wih = wih_ref[...]
    whh = whh_ref[...]
    bih = bih_ref[...]
    bhh = bhh_ref[...]
    h = jnp.zeros((BG, OUT), jnp.float32)
    for t in range(W):
        xt = dinv_ref[t] * (acc2_ref[t, 0] + acc2_ref[t, 1] + h2_ref[t]) + b2_ref[t]
        gi = jnp.dot(xt, wih, preferred_element_type=jnp.float32) + bih
        gh = jnp.dot(h, whh, preferred_element_type=jnp.float32) + bhh
        r = jax.nn.sigmoid(gi[:, :OUT] + gh[:, :OUT])
        z = jax.nn.sigmoid(gi[:, OUT:2 * OUT] + gh[:, OUT:2 * OUT])
        n = jnp.tanh(gi[:, 2 * OUT:] + r * gh[:, 2 * OUT:])
        h = (1.0 - z) * n + z * h
    out_ref[...] = h


def _prep_edges(src, dst):
    woff = (jnp.arange(W, dtype=jnp.int32) * N)[:, None, None]

    per1 = E // NS
    pad1 = ET1 - per1
    psrc1 = (jnp.arange(pad1, dtype=jnp.int32) * 7) % N
    pdst1 = N + (jnp.arange(pad1, dtype=jnp.int32) % (NPAD - N))
    s1 = jnp.concatenate(
        [src.reshape(W, NS, per1), jnp.broadcast_to(psrc1, (W, NS, pad1))], -1) + woff
    s1 = jnp.stack([s1, s1 + W * N])
    s1 = jnp.transpose(s1, (1, 0, 2, 3)).reshape(W, NC, NS, K1, 128)
    d1 = jnp.concatenate(
        [dst.reshape(W, NS, per1), jnp.broadcast_to(pdst1, (W, NS, pad1))], -1)
    d1 = jnp.broadcast_to(d1[:, None], (W, NC, NS, ET1)).reshape(W, NC, NS, K1, 128)

    per2 = E // NWK
    pad2 = ET2 - per2
    psrc2 = (jnp.arange(pad2, dtype=jnp.int32) * 11) % N
    pdst2 = N + (jnp.arange(pad2, dtype=jnp.int32) % (NPAD - N))
    s2 = jnp.concatenate(
        [src.reshape(W, NWK, per2), jnp.broadcast_to(psrc2, (W, NWK, pad2))], -1) + woff
    s2 = s2.reshape(W, NC, NS, K2, 128)
    d2 = jnp.concatenate(
        [dst.reshape(W, NWK, per2), jnp.broadcast_to(pdst2, (W, NWK, pad2))], -1)
    d2 = d2.reshape(W, NC, NS, K2, 128)
    return s1, d1, s2, d2


def _prep_emb_idx(ids):
    idsp = jnp.transpose(ids, (2, 0, 1))
    padn = (jnp.arange(NPAD - N, dtype=jnp.int32) * 13) % CAT_DIM
    idsp = jnp.concatenate(
        [idsp, jnp.broadcast_to(padn, (NCAT, W, NPAD - N))], -1)
    eidx = idsp + (jnp.arange(NCAT, dtype=jnp.int32) * CAT_DIM)[:, None, None]
    return eidx.reshape(NC, NS, KE, 128)


def kernel(x, edge_index, emb_c0, emb_c1, emb_c2, gcn1_w, gcn1_b, gcn2_w, gcn2_b,
           w_ih, w_hh, b_ih, b_hh):
    ids = x[:, :, :NCAT].astype(jnp.int32)
    xnum = x[:, :, NCAT:]
    src = edge_index[:, 0, :]
    dst = edge_index[:, 1, :]

    s1, d1, s2, d2 = _prep_edges(src, dst)
    eidx = _prep_emb_idx(ids)
    emb_all = jnp.concatenate([emb_c0, emb_c1, emb_c2], axis=0)
    zeros_hbm = jnp.zeros((NPAD, OUT), jnp.float32)

    cat = _emb_gather(emb_all, eidx).reshape(NCAT, W, NPAD, OUT)
    degp = _degree(d2)

    w1c = gcn1_w[:, :NCAT * OUT].reshape(W, NCAT, OUT, 2 * OUT)
    w1n = gcn1_w[:, NCAT * OUT:]
    nblk = N // BN
    h1s, dinv = pl.pallas_call(
        _dense1_body,
        grid=(W, nblk),
        in_specs=[
            pl.BlockSpec((NCAT, 1, BN, OUT), lambda w, i: (0, w, i, 0)),
            pl.BlockSpec((1, BN, NNUM), lambda w, i: (w, i, 0)),
            pl.BlockSpec((1, NC, BN, 1), lambda w, i: (w, 0, i, 0)),
            pl.BlockSpec((1, NCAT, OUT, 2 * OUT), lambda w, i: (w, 0, 0, 0)),
            pl.BlockSpec((1, NNUM, 2 * OUT), lambda w, i: (w, 0, 0)),
        ],
        out_specs=[
            pl.BlockSpec((NC, 1, BN, OUT), lambda w, i: (0, w, i, 0)),
            pl.BlockSpec((1, BN, 1), lambda w, i: (w, i, 0)),
        ],
        out_shape=[
            jax.ShapeDtypeStruct((NC, W, N, OUT), jnp.float32),
            jax.ShapeDtypeStruct((W, N, 1), jnp.float32),
        ],
    )(cat, xnum, degp.reshape(W, NC, NPAD, 1), w1c, w1n)

    acc1 = _conv1(h1s.reshape(NC * W * N, OUT), s1, d1, zeros_hbm)

    b1 = gcn1_b.reshape(W, NC, OUT)
    w2 = gcn2_w.reshape(W, NC, OUT, OUT)
    h2 = pl.pallas_call(
        _dense2_body,
        grid=(W, nblk),
        in_specs=[
            pl.BlockSpec((1, NC, BN, OUT), lambda w, i: (w, 0, i, 0)),
            pl.BlockSpec((NC, 1, BN, OUT), lambda w, i: (0, w, i, 0)),
            pl.BlockSpec((1, BN, 1), lambda w, i: (w, i, 0)),
            pl.BlockSpec((1, NC, OUT), lambda w, i: (w, 0, 0)),
            pl.BlockSpec((1, NC, OUT, OUT), lambda w, i: (w, 0, 0, 0)),
        ],
        out_specs=pl.BlockSpec((1, BN, OUT), lambda w, i: (w, i, 0)),
        out_shape=jax.ShapeDtypeStruct((W, N, OUT), jnp.float32),
    )(acc1, h1s, dinv, b1, w2)

    acc2 = _conv2(h2.reshape(W * N, OUT), s2, d2, zeros_hbm)

    nblk_g = N // BG
    out = pl.pallas_call(
        _gru_body,
        grid=(nblk_g,),
        in_specs=[
            pl.BlockSpec((W, NC, BG, OUT), lambda i: (0, 0, i, 0)),
            pl.BlockSpec((W, BG, OUT), lambda i: (0, i, 0)),
            pl.BlockSpec((W, BG, 1), lambda i: (0, i, 0)),
            pl.BlockSpec((W, OUT), lambda i: (0, 0)),
            pl.BlockSpec((OUT, 3 * OUT), lambda i: (0, 0)),
            pl.BlockSpec((OUT, 3 * OUT), lambda i: (0, 0)),
            pl.BlockSpec((1, 3 * OUT), lambda i: (0, 0)),
            pl.BlockSpec((1, 3 * OUT), lambda i: (0, 0)),
        ],
        out_specs=pl.BlockSpec((BG, OUT), lambda i: (i, 0)),
        out_shape=jax.ShapeDtypeStruct((N, OUT), jnp.float32),
    )(acc2, h2, dinv, gcn2_b, w_ih.T, w_hh.T,
      b_ih.reshape(1, -1), b_hh.reshape(1, -1))
    return out


# speedup vs baseline: 15.4155x; 1.3391x over previous
import functools

import jax
import jax.numpy as jnp
from jax import lax
from jax.experimental import pallas as pl
from jax.experimental.pallas import tpu as pltpu
from jax.experimental.pallas import tpu_sc as plsc

N = 10000
W = 12
OUT = 128
NCAT = 3
NNUM = 16
CAT_DIM = 1000
E = 160000

NPAD = 10240
NC, NS = 2, 16
NWK = NC * NS
STRIPE = NPAD // NS

K1 = 80
ET1 = K1 * 128
K2 = 40
ET2 = K2 * 128
KG = 8
KE = 90
EROWS = NCAT * W * NPAD

BN = 2000
BG = 1000

_mesh = plsc.VectorSubcoreMesh(core_axis_name="c", subcore_axis_name="s")


@functools.partial(
    pl.kernel,
    out_type=jax.ShapeDtypeStruct((EROWS, OUT), jnp.float32),
    mesh=_mesh,
    scratch_types=[
        pltpu.VMEM((KE, 128), jnp.int32),
        pltpu.VMEM((2, 128, OUT), jnp.float32),
        pltpu.SemaphoreType.DMA((2,)),
    ],
)
def _emb_gather(emb_hbm, eidx_hbm, out_hbm, idx_v, rows_v, gsem):
    c = lax.axis_index("c")
    s = lax.axis_index("s")
    base = (c * NS + s) * (KE * 128)
    pltpu.sync_copy(eidx_hbm.at[c, s], idx_v)
    pltpu.async_copy(emb_hbm.at[idx_v.at[0]], rows_v.at[0], gsem.at[0])

    def body(j, carry):
        b = lax.rem(j, 2)
        nb = lax.rem(j + 1, 2)

        @pl.when(j < KE - 1)
        def _():
            pltpu.async_copy(emb_hbm.at[idx_v.at[j + 1]], rows_v.at[nb],
                             gsem.at[nb])

        pltpu.make_async_copy(emb_hbm.at[idx_v.at[j]], rows_v.at[b],
                              gsem.at[b]).wait()
        pltpu.sync_copy(rows_v.at[b], out_hbm.at[pl.ds(base + j * 128, 128)])
        return carry

    lax.fori_loop(0, KE, body, 0)


@functools.partial(
    pl.kernel,
    out_type=jax.ShapeDtypeStruct((W, NC, NPAD), jnp.float32),
    mesh=_mesh,
    scratch_types=[
        pltpu.VMEM((K2, 128), jnp.int32),
        pltpu.VMEM((128,), jnp.float32),
        pltpu.VMEM((STRIPE,), jnp.float32),
        pltpu.VMEM_SHARED((NPAD,), jnp.float32),
    ],
)
def _degree(didx_hbm, out_hbm, idx_v, ones_v, zeros_v, deg_sp):
    c = lax.axis_index("c")
    s = lax.axis_index("s")

    def fill(i, carry):
        ones_v[pl.ds(i * 16, 16)] = jnp.full((16,), 1.0, jnp.float32)
        return carry

    lax.fori_loop(0, 8, fill, 0)

    def zfill(i, carry):
        zeros_v[pl.ds(i * 16, 16)] = jnp.zeros((16,), jnp.float32)
        return carry

    lax.fori_loop(0, STRIPE // 16, zfill, 0)

    def win(w, carry):
        pltpu.sync_copy(zeros_v, deg_sp.at[pl.ds(s * STRIPE, STRIPE)])
        plsc.subcore_barrier()
        pltpu.sync_copy(didx_hbm.at[w, c, s], idx_v)

        def body(j, carry2):
            pltpu.sync_copy(ones_v, deg_sp.at[idx_v.at[j]], add=True)
            return carry2

        lax.fori_loop(0, K2, body, 0)
        plsc.subcore_barrier()
        pltpu.sync_copy(deg_sp.at[pl.ds(s * STRIPE, STRIPE)],
                        out_hbm.at[w, c, pl.ds(s * STRIPE, STRIPE)])
        plsc.subcore_barrier()
        return carry

    lax.fori_loop(0, W, win, 0)


def _make_conv(k_chunks):
    @functools.partial(
        pl.kernel,
        out_type=jax.ShapeDtypeStruct((W, NC, NPAD, OUT), jnp.float32),
        mesh=_mesh,
        scratch_types=[
            pltpu.VMEM((2, KG, 128), jnp.int32),
            pltpu.VMEM((2, KG, 128), jnp.int32),
            pltpu.VMEM((2, 128, OUT), jnp.float32),
            pltpu.VMEM_SHARED((NPAD, OUT), jnp.float32),
            pltpu.SemaphoreType.DMA((2,)),
            pltpu.SemaphoreType.DMA((2,)),
        ],
    )
    def conv(table_hbm, sidx_hbm, didx_hbm, zeros_hbm, out_hbm,
             sidx_v, didx_v, rows_v, acc_sp, gsem, isem):
        c = lax.axis_index("c")
        s = lax.axis_index("s")
        n_grp = k_chunks // KG

        def win(w, carry):
            pltpu.sync_copy(zeros_hbm.at[pl.ds(s * STRIPE, STRIPE)],
                            acc_sp.at[pl.ds(s * STRIPE, STRIPE)])
            plsc.subcore_barrier()
            pltpu.async_copy(sidx_hbm.at[w, c, s, pl.ds(0, KG)], sidx_v.at[0],
                             isem.at[0])
            pltpu.async_copy(didx_hbm.at[w, c, s, pl.ds(0, KG)], didx_v.at[0],
                             isem.at[1])
            pltpu.make_async_copy(sidx_hbm.at[w, c, s, pl.ds(0, KG)],
                                  sidx_v.at[0], isem.at[0]).wait()
            pltpu.make_async_copy(didx_hbm.at[w, c, s, pl.ds(0, KG)],
                                  didx_v.at[0], isem.at[1]).wait()
            pltpu.async_copy(table_hbm.at[sidx_v.at[0, 0]], rows_v.at[0],
                             gsem.at[0])

            def grp(g, carry1):
                gb = lax.rem(g, 2)
                gn = lax.rem(g + 1, 2)

                @pl.when(g < n_grp - 1)
                def _():
                    pltpu.async_copy(
                        sidx_hbm.at[w, c, s, pl.ds((g + 1) * KG, KG)],
                        sidx_v.at[gn], isem.at[0])
                    pltpu.async_copy(
                        didx_hbm.at[w, c, s, pl.ds((g + 1) * KG, KG)],
                        didx_v.at[gn], isem.at[1])

                def body(jj, carry2):
                    j = g * KG + jj
                    b = lax.rem(j, 2)
                    nb = lax.rem(j + 1, 2)

                    @pl.when(jj < KG - 1)
                    def _():
                        pltpu.async_copy(table_hbm.at[sidx_v.at[gb, jj + 1]],
                                         rows_v.at[nb], gsem.at[nb])

                    pltpu.make_async_copy(table_hbm.at[sidx_v.at[gb, jj]],
                                          rows_v.at[b], gsem.at[b]).wait()
                    pltpu.sync_copy(rows_v.at[b], acc_sp.at[didx_v.at[gb, jj]],
                                    add=True)
                    return carry2

                lax.fori_loop(0, KG, body, 0)

                @pl.when(g < n_grp - 1)
                def _():
                    pltpu.make_async_copy(
                        sidx_hbm.at[w, c, s, pl.ds((g + 1) * KG, KG)],
                        sidx_v.at[gn], isem.at[0]).wait()
                    pltpu.make_async_copy(
                        didx_hbm.at[w, c, s, pl.ds((g + 1) * KG, KG)],
                        didx_v.at[gn], isem.at[1]).wait()
                    j0 = (g + 1) * KG
                    pltpu.async_copy(table_hbm.at[sidx_v.at[gn, 0]],
                                     rows_v.at[lax.rem(j0, 2)],
                                     gsem.at[lax.rem(j0, 2)])
                return carry1

            lax.fori_loop(0, n_grp, grp, 0)
            plsc.subcore_barrier()
            pltpu.sync_copy(acc_sp.at[pl.ds(s * STRIPE, STRIPE)],
                            out_hbm.at[w, c, pl.ds(s * STRIPE, STRIPE)])
            plsc.subcore_barrier()
            return carry

        lax.fori_loop(0, W, win, 0)

    return conv


_conv1 = _make_conv(K1)
_conv2 = _make_conv(K2)


def _dense1_body(cat_ref, xnum_ref, deg_ref, w1c_ref, w1n_ref, h1s_ref, dinv_ref):
    deg = jnp.sum(deg_ref[0, :, :, 0], axis=0) + 1.0
    dinv = lax.rsqrt(deg)
    acc = jnp.dot(cat_ref[0, 0], w1c_ref[0, 0], preferred_element_type=jnp.float32)
    acc = acc + jnp.dot(cat_ref[1, 0], w1c_ref[0, 1], preferred_element_type=jnp.float32)
    acc = acc + jnp.dot(cat_ref[2, 0], w1c_ref[0, 2], preferred_element_type=jnp.float32)
    acc = acc + jnp.dot(xnum_ref[0], w1n_ref[0], preferred_element_type=jnp.float32)
    h1s = acc * dinv[:, None]
    h1s_ref[0, 0] = h1s[:, :OUT]
    h1s_ref[1, 0] = h1s[:, OUT:]
    dinv_ref[0, :, 0] = dinv


def _dense2_body(acc1_ref, h1s_ref, dinv_ref, b1_ref, w2_ref, h2_ref):
    dinv = dinv_ref[0, :, 0]
    res = None
    for half in range(2):
        pre = acc1_ref[0, half] + h1s_ref[half, 0]
        h1h = jnp.maximum(dinv[:, None] * pre + b1_ref[0, half], 0.0)
        d = jnp.dot(h1h, w2_ref[0, half], preferred_element_type=jnp.float32)
        res = d if res is None else res + d
    h2_ref[0] = res * dinv[:, None]


def _gru_body(acc2_ref, h2_ref, dinv_ref, b2_ref, wih_ref, whh_ref,
              bih_ref, bhh_ref, out_ref):
    wih = wih_ref[...]
    whh = whh_ref[...]
    bih = bih_ref[...]
    bhh = bhh_ref[...]
    h = jnp.zeros((BG, OUT), jnp.float32)
    for t in range(W):
        xt = dinv_ref[t] * (acc2_ref[t, 0] + acc2_ref[t, 1] + h2_ref[t]) + b2_ref[t]
        gi = jnp.dot(xt, wih, preferred_element_type=jnp.float32) + bih
        gh = jnp.dot(h, whh, preferred_element_type=jnp.float32) + bhh
        r = jax.nn.sigmoid(gi[:, :OUT] + gh[:, :OUT])
        z = jax.nn.sigmoid(gi[:, OUT:2 * OUT] + gh[:, OUT:2 * OUT])
        n = jnp.tanh(gi[:, 2 * OUT:] + r * gh[:, 2 * OUT:])
        h = (1.0 - z) * n + z * h
    out_ref[...] = h


def _prep_edges(src, dst):
    woff = (jnp.arange(W, dtype=jnp.int32) * N)[:, None, None]

    per1 = E // NS
    pad1 = ET1 - per1
    psrc1 = (jnp.arange(pad1, dtype=jnp.int32) * 7) % N
    pdst1 = N + (jnp.arange(pad1, dtype=jnp.int32) % (NPAD - N))
    s1 = jnp.concatenate(
        [src.reshape(W, NS, per1), jnp.broadcast_to(psrc1, (W, NS, pad1))], -1) + woff
    s1 = jnp.stack([s1, s1 + W * N])
    s1 = jnp.transpose(s1, (1, 0, 2, 3)).reshape(W, NC, NS, K1, 128)
    d1 = jnp.concatenate(
        [dst.reshape(W, NS, per1), jnp.broadcast_to(pdst1, (W, NS, pad1))], -1)
    d1 = jnp.broadcast_to(d1[:, None], (W, NC, NS, ET1)).reshape(W, NC, NS, K1, 128)

    per2 = E // NWK
    pad2 = ET2 - per2
    psrc2 = (jnp.arange(pad2, dtype=jnp.int32) * 11) % N
    pdst2 = N + (jnp.arange(pad2, dtype=jnp.int32) % (NPAD - N))
    s2 = jnp.concatenate(
        [src.reshape(W, NWK, per2), jnp.broadcast_to(psrc2, (W, NWK, pad2))], -1) + woff
    s2 = s2.reshape(W, NC, NS, K2, 128)
    d2 = jnp.concatenate(
        [dst.reshape(W, NWK, per2), jnp.broadcast_to(pdst2, (W, NWK, pad2))], -1)
    d2 = d2.reshape(W, NC, NS, K2, 128)
    return s1, d1, s2, d2


def _prep_emb_idx(ids):
    idsp = jnp.transpose(ids, (2, 0, 1))
    padn = (jnp.arange(NPAD - N, dtype=jnp.int32) * 13) % CAT_DIM
    idsp = jnp.concatenate(
        [idsp, jnp.broadcast_to(padn, (NCAT, W, NPAD - N))], -1)
    eidx = idsp + (jnp.arange(NCAT, dtype=jnp.int32) * CAT_DIM)[:, None, None]
    return eidx.reshape(NC, NS, KE, 128)


def kernel(x, edge_index, emb_c0, emb_c1, emb_c2, gcn1_w, gcn1_b, gcn2_w, gcn2_b,
           w_ih, w_hh, b_ih, b_hh):
    ids = x[:, :, :NCAT].astype(jnp.int32)
    xnum = x[:, :, NCAT:]
    src = edge_index[:, 0, :]
    dst = edge_index[:, 1, :]

    s1, d1, s2, d2 = _prep_edges(src, dst)
    eidx = _prep_emb_idx(ids)
    emb_all = jnp.concatenate([emb_c0, emb_c1, emb_c2], axis=0)
    zeros_hbm = jnp.zeros((NPAD, OUT), jnp.float32)

    cat = _emb_gather(emb_all, eidx).reshape(NCAT, W, NPAD, OUT)
    degp = _degree(d2)

    w1c = gcn1_w[:, :NCAT * OUT].reshape(W, NCAT, OUT, 2 * OUT)
    w1n = gcn1_w[:, NCAT * OUT:]
    nblk = N // BN
    h1s, dinv = pl.pallas_call(
        _dense1_body,
        grid=(W, nblk),
        in_specs=[
            pl.BlockSpec((NCAT, 1, BN, OUT), lambda w, i: (0, w, i, 0)),
            pl.BlockSpec((1, BN, NNUM), lambda w, i: (w, i, 0)),
            pl.BlockSpec((1, NC, BN, 1), lambda w, i: (w, 0, i, 0)),
            pl.BlockSpec((1, NCAT, OUT, 2 * OUT), lambda w, i: (w, 0, 0, 0)),
            pl.BlockSpec((1, NNUM, 2 * OUT), lambda w, i: (w, 0, 0)),
        ],
        out_specs=[
            pl.BlockSpec((NC, 1, BN, OUT), lambda w, i: (0, w, i, 0)),
            pl.BlockSpec((1, BN, 1), lambda w, i: (w, i, 0)),
        ],
        out_shape=[
            jax.ShapeDtypeStruct((NC, W, N, OUT), jnp.float32),
            jax.ShapeDtypeStruct((W, N, 1), jnp.float32),
        ],
    )(cat, xnum, degp.reshape(W, NC, NPAD, 1), w1c, w1n)

    acc1 = _conv1(h1s.reshape(NC * W * N, OUT), s1, d1, zeros_hbm)

    b1 = gcn1_b.reshape(W, NC, OUT)
    w2 = gcn2_w.reshape(W, NC, OUT, OUT)
    h2 = pl.pallas_call(
        _dense2_body,
        grid=(W, nblk),
        in_specs=[
            pl.BlockSpec((1, NC, BN, OUT), lambda w, i: (w, 0, i, 0)),
            pl.BlockSpec((NC, 1, BN, OUT), lambda w, i: (0, w, i, 0)),
            pl.BlockSpec((1, BN, 1), lambda w, i: (w, i, 0)),
            pl.BlockSpec((1, NC, OUT), lambda w, i: (w, 0, 0)),
            pl.BlockSpec((1, NC, OUT, OUT), lambda w, i: (w, 0, 0, 0)),
        ],
        out_specs=pl.BlockSpec((1, BN, OUT), lambda w, i: (w, i, 0)),
        out_shape=jax.ShapeDtypeStruct((W, N, OUT), jnp.float32),
    )(acc1, h1s, dinv, b1, w2)

    acc2 = _conv2(h2.reshape(W * N, OUT), s2, d2, zeros_hbm)

    nblk_g = N // BG
    out = pl.pallas_call(
        _gru_body,
        grid=(nblk_g,),
        in_specs=[
            pl.BlockSpec((W, NC, BG, OUT), lambda i: (0, 0, i, 0)),
            pl.BlockSpec((W, BG, OUT), lambda i: (0, i, 0)),
            pl.BlockSpec((W, BG, 1), lambda i: (0, i, 0)),
            pl.BlockSpec((W, OUT), lambda i: (0, 0)),
            pl.BlockSpec((OUT, 3 * OUT), lambda i: (0, 0)),
            pl.BlockSpec((OUT, 3 * OUT), lambda i: (0, 0)),
            pl.BlockSpec((1, 3 * OUT), lambda i: (0, 0)),
            pl.BlockSpec((1, 3 * OUT), lambda i: (0, 0)),
        ],
        out_specs=pl.BlockSpec((BG, OUT), lambda i: (i, 0)),
        out_shape=jax.ShapeDtypeStruct((N, OUT), jnp.float32),
    )(acc2, h2, dinv, gcn2_b, w_ih.T, w_hh.T,
      b_ih.reshape(1, -1), b_hh.reshape(1, -1))
    return out


# speedup vs baseline: 16.6912x; 1.0828x over previous
import functools

import jax
import jax.numpy as jnp
from jax import lax
from jax.experimental import pallas as pl
from jax.experimental.pallas import tpu as pltpu
from jax.experimental.pallas import tpu_sc as plsc

N = 10000
W = 12
OUT = 128
NCAT = 3
NNUM = 16
CAT_DIM = 1000
E = 160000

NPAD = 10240
NC, NS = 2, 16
NWK = NC * NS
STRIPE = NPAD // NS

CH = 64
K1 = 160
ET1 = K1 * CH
K2 = 80
ET2 = K2 * CH
KG = 16
NB = 4
KE = 90
EROWS = NCAT * W * NPAD

BN = 2000
BG = 1000

_mesh = plsc.VectorSubcoreMesh(core_axis_name="c", subcore_axis_name="s")


@functools.partial(
    pl.kernel,
    out_type=jax.ShapeDtypeStruct((EROWS, OUT), jnp.float32),
    mesh=_mesh,
    scratch_types=[
        pltpu.VMEM((KE, 128), jnp.int32),
        pltpu.VMEM((2, 128, OUT), jnp.float32),
        pltpu.SemaphoreType.DMA((2,)),
    ],
)
def _emb_gather(emb_hbm, eidx_hbm, out_hbm, idx_v, rows_v, gsem):
    c = lax.axis_index("c")
    s = lax.axis_index("s")
    base = (c * NS + s) * (KE * 128)
    pltpu.sync_copy(eidx_hbm.at[c, s], idx_v)
    pltpu.async_copy(emb_hbm.at[idx_v.at[0]], rows_v.at[0], gsem.at[0])

    def body(j, carry):
        b = lax.rem(j, 2)
        nb = lax.rem(j + 1, 2)

        @pl.when(j < KE - 1)
        def _():
            pltpu.async_copy(emb_hbm.at[idx_v.at[j + 1]], rows_v.at[nb],
                             gsem.at[nb])

        pltpu.make_async_copy(emb_hbm.at[idx_v.at[j]], rows_v.at[b],
                              gsem.at[b]).wait()
        pltpu.sync_copy(rows_v.at[b], out_hbm.at[pl.ds(base + j * 128, 128)])
        return carry

    lax.fori_loop(0, KE, body, 0)


@functools.partial(
    pl.kernel,
    out_type=jax.ShapeDtypeStruct((W, NC, NPAD), jnp.float32),
    mesh=_mesh,
    scratch_types=[
        pltpu.VMEM((ET2 // 128, 128), jnp.int32),
        pltpu.VMEM((128,), jnp.float32),
        pltpu.VMEM((STRIPE,), jnp.float32),
        pltpu.VMEM_SHARED((NPAD,), jnp.float32),
    ],
)
def _degree(didx_hbm, out_hbm, idx_v, ones_v, zeros_v, deg_sp):
    c = lax.axis_index("c")
    s = lax.axis_index("s")

    def fill(i, carry):
        ones_v[pl.ds(i * 16, 16)] = jnp.full((16,), 1.0, jnp.float32)
        return carry

    lax.fori_loop(0, 8, fill, 0)

    def zfill(i, carry):
        zeros_v[pl.ds(i * 16, 16)] = jnp.zeros((16,), jnp.float32)
        return carry

    lax.fori_loop(0, STRIPE // 16, zfill, 0)

    def win(w, carry):
        pltpu.sync_copy(zeros_v, deg_sp.at[pl.ds(s * STRIPE, STRIPE)])
        plsc.subcore_barrier()
        pltpu.sync_copy(didx_hbm.at[w, c, s], idx_v)

        def body(j, carry2):
            pltpu.sync_copy(ones_v, deg_sp.at[idx_v.at[j]], add=True)
            return carry2

        lax.fori_loop(0, ET2 // 128, body, 0)
        plsc.subcore_barrier()
        pltpu.sync_copy(deg_sp.at[pl.ds(s * STRIPE, STRIPE)],
                        out_hbm.at[w, c, pl.ds(s * STRIPE, STRIPE)])
        plsc.subcore_barrier()
        return carry

    lax.fori_loop(0, W, win, 0)


def _make_conv(k_chunks):
    @functools.partial(
        pl.kernel,
        out_type=jax.ShapeDtypeStruct((W, NC, NPAD, OUT), jnp.float32),
        mesh=_mesh,
        scratch_types=[
            pltpu.VMEM((2, KG, CH), jnp.int32),
            pltpu.VMEM((2, KG, CH), jnp.int32),
            pltpu.VMEM((NB, CH, OUT), jnp.float32),
            pltpu.VMEM_SHARED((NPAD, OUT), jnp.float32),
            pltpu.SemaphoreType.DMA((NB,)),
            pltpu.SemaphoreType.DMA((NB,)),
            pltpu.SemaphoreType.DMA((2,)),
        ],
    )
    def conv(table_hbm, sidx_hbm, didx_hbm, zeros_hbm, out_hbm,
             sidx_v, didx_v, rows_v, acc_sp, gsem, ssem, isem):
        c = lax.axis_index("c")
        s = lax.axis_index("s")
        n_grp = k_chunks // KG

        def _wait_scatter(b):
            pltpu.make_async_copy(rows_v.at[b], acc_sp.at[didx_v.at[0, 0]],
                                  ssem.at[b]).wait()

        pltpu.sync_copy(zeros_hbm.at[pl.ds(s * STRIPE, STRIPE)],
                        acc_sp.at[pl.ds(s * STRIPE, STRIPE)])
        plsc.subcore_barrier()

        def win(w, carry):
            pltpu.sync_copy(sidx_hbm.at[w, c, s, pl.ds(0, KG)], sidx_v.at[0])
            pltpu.sync_copy(didx_hbm.at[w, c, s, pl.ds(0, KG)], didx_v.at[0])
            for t in range(3):
                pltpu.async_copy(table_hbm.at[sidx_v.at[0, t]], rows_v.at[t],
                                 gsem.at[t])

            def grp(g, carry1):
                gb = lax.rem(g, 2)
                gn = lax.rem(g + 1, 2)

                def body(jj, carry2):
                    j = g * KG + jj
                    b = lax.rem(j, NB)
                    pltpu.make_async_copy(table_hbm.at[sidx_v.at[gb, jj]],
                                          rows_v.at[b], gsem.at[b]).wait()
                    pltpu.async_copy(rows_v.at[b], acc_sp.at[didx_v.at[gb, jj]],
                                     ssem.at[b], add=True)

                    @pl.when(jnp.logical_and(jj == 1, g < n_grp - 1))
                    def _():
                        pltpu.async_copy(
                            sidx_hbm.at[w, c, s, pl.ds((g + 1) * KG, KG)],
                            sidx_v.at[gn], isem.at[0])
                        pltpu.async_copy(
                            didx_hbm.at[w, c, s, pl.ds((g + 1) * KG, KG)],
                            didx_v.at[gn], isem.at[1])

                    @pl.when(jj < KG - 3)
                    def _():
                        nb4 = lax.rem(j + 3, NB)

                        @pl.when(j + 3 >= NB)
                        def _():
                            _wait_scatter(nb4)

                        pltpu.async_copy(table_hbm.at[sidx_v.at[gb, jj + 3]],
                                         rows_v.at[nb4], gsem.at[nb4])
                    return carry2

                lax.fori_loop(0, KG, body, 0)

                @pl.when(g < n_grp - 1)
                def _():
                    pltpu.make_async_copy(
                        sidx_hbm.at[w, c, s, pl.ds((g + 1) * KG, KG)],
                        sidx_v.at[gn], isem.at[0]).wait()
                    pltpu.make_async_copy(
                        didx_hbm.at[w, c, s, pl.ds((g + 1) * KG, KG)],
                        didx_v.at[gn], isem.at[1]).wait()
                    for t in range(3):
                        j0 = (g + 1) * KG + t
                        b0 = lax.rem(j0, NB)
                        _wait_scatter(b0)
                        pltpu.async_copy(table_hbm.at[sidx_v.at[gn, t]],
                                         rows_v.at[b0], gsem.at[b0])
                return carry1

            lax.fori_loop(0, n_grp, grp, 0)
            for b in range(NB):
                _wait_scatter(b)
            plsc.subcore_barrier()
            pltpu.sync_copy(acc_sp.at[pl.ds(s * STRIPE, STRIPE)],
                            out_hbm.at[w, c, pl.ds(s * STRIPE, STRIPE)])
            pltpu.sync_copy(zeros_hbm.at[pl.ds(s * STRIPE, STRIPE)],
                            acc_sp.at[pl.ds(s * STRIPE, STRIPE)])
            plsc.subcore_barrier()
            return carry

        lax.fori_loop(0, W, win, 0)

    return conv


_conv1 = _make_conv(K1)
_conv2 = _make_conv(K2)


def _dense1_body(cat_ref, xnum_ref, deg_ref, w1c_ref, w1n_ref, h1s_ref, dinv_ref):
    deg = jnp.sum(deg_ref[0, :, :, 0], axis=0) + 1.0
    dinv = lax.rsqrt(deg)
    acc = jnp.dot(cat_ref[0, 0], w1c_ref[0, 0], preferred_element_type=jnp.float32)
    acc = acc + jnp.dot(cat_ref[1, 0], w1c_ref[0, 1], preferred_element_type=jnp.float32)
    acc = acc + jnp.dot(cat_ref[2, 0], w1c_ref[0, 2], preferred_element_type=jnp.float32)
    acc = acc + jnp.dot(xnum_ref[0], w1n_ref[0], preferred_element_type=jnp.float32)
    h1s = acc * dinv[:, None]
    h1s_ref[0, 0] = h1s[:, :OUT]
    h1s_ref[1, 0] = h1s[:, OUT:]
    dinv_ref[0, :, 0] = dinv


def _dense2_body(acc1_ref, h1s_ref, dinv_ref, b1_ref, w2_ref, h2_ref):
    dinv = dinv_ref[0, :, 0]
    res = None
    for half in range(2):
        pre = acc1_ref[0, half] + h1s_ref[half, 0]
        h1h = jnp.maximum(dinv[:, None] * pre + b1_ref[0, half], 0.0)
        d = jnp.dot(h1h, w2_ref[0, half], preferred_element_type=jnp.float32)
        res = d if res is None else res + d
    h2_ref[0] = res * dinv[:, None]


def _gru_body(acc2_ref, h2_ref, dinv_ref, b2_ref, wih_ref, whh_ref,
              bih_ref, bhh_ref, out_ref):
    wih = wih_ref[...]
    whh = whh_ref[...]
    bih = bih_ref[...]
    bhh = bhh_ref[...]
    h = jnp.zeros((BG, OUT), jnp.float32)
    for t in range(W):
        xt = dinv_ref[t] * (acc2_ref[t, 0] + acc2_ref[t, 1] + h2_ref[t]) + b2_ref[t]
        gi = jnp.dot(xt, wih, preferred_element_type=jnp.float32) + bih
        gh = jnp.dot(h, whh, preferred_element_type=jnp.float32) + bhh
        r = jax.nn.sigmoid(gi[:, :OUT] + gh[:, :OUT])
        z = jax.nn.sigmoid(gi[:, OUT:2 * OUT] + gh[:, OUT:2 * OUT])
        n = jnp.tanh(gi[:, 2 * OUT:] + r * gh[:, 2 * OUT:])
        h = (1.0 - z) * n + z * h
    out_ref[...] = h


def _prep_edges(src, dst):
    woff = (jnp.arange(W, dtype=jnp.int32) * N)[:, None, None]

    per1 = E // NS
    pad1 = ET1 - per1
    psrc1 = (jnp.arange(pad1, dtype=jnp.int32) * 7) % N
    pdst1 = N + (jnp.arange(pad1, dtype=jnp.int32) % (NPAD - N))
    s1 = jnp.concatenate(
        [src.reshape(W, NS, per1), jnp.broadcast_to(psrc1, (W, NS, pad1))], -1) + woff
    s1 = jnp.stack([s1, s1 + W * N])
    s1 = jnp.transpose(s1, (1, 0, 2, 3)).reshape(W, NC, NS, K1, CH)
    d1 = jnp.concatenate(
        [dst.reshape(W, NS, per1), jnp.broadcast_to(pdst1, (W, NS, pad1))], -1)
    d1 = jnp.broadcast_to(d1[:, None], (W, NC, NS, ET1)).reshape(W, NC, NS, K1, CH)

    per2 = E // NWK
    pad2 = ET2 - per2
    psrc2 = (jnp.arange(pad2, dtype=jnp.int32) * 11) % N
    pdst2 = N + (jnp.arange(pad2, dtype=jnp.int32) % (NPAD - N))
    s2 = jnp.concatenate(
        [src.reshape(W, NWK, per2), jnp.broadcast_to(psrc2, (W, NWK, pad2))], -1) + woff
    s2 = s2.reshape(W, NC, NS, K2, CH)
    d2 = jnp.concatenate(
        [dst.reshape(W, NWK, per2), jnp.broadcast_to(pdst2, (W, NWK, pad2))], -1)
    d2 = d2.reshape(W, NC, NS, K2, CH)
    return s1, d1, s2, d2


def _prep_emb_idx(ids):
    idsp = jnp.transpose(ids, (2, 0, 1))
    padn = (jnp.arange(NPAD - N, dtype=jnp.int32) * 13) % CAT_DIM
    idsp = jnp.concatenate(
        [idsp, jnp.broadcast_to(padn, (NCAT, W, NPAD - N))], -1)
    eidx = idsp + (jnp.arange(NCAT, dtype=jnp.int32) * CAT_DIM)[:, None, None]
    return eidx.reshape(NC, NS, KE, 128)


def kernel(x, edge_index, emb_c0, emb_c1, emb_c2, gcn1_w, gcn1_b, gcn2_w, gcn2_b,
           w_ih, w_hh, b_ih, b_hh):
    ids = x[:, :, :NCAT].astype(jnp.int32)
    xnum = x[:, :, NCAT:]
    src = edge_index[:, 0, :]
    dst = edge_index[:, 1, :]

    s1, d1, s2, d2 = _prep_edges(src, dst)
    eidx = _prep_emb_idx(ids)
    emb_all = jnp.concatenate([emb_c0, emb_c1, emb_c2], axis=0)
    zeros_hbm = jnp.zeros((NPAD, OUT), jnp.float32)

    cat = _emb_gather(emb_all, eidx).reshape(NCAT, W, NPAD, OUT)
    degp = _degree(d2.reshape(W, NC, NS, ET2 // 128, 128))

    w1c = gcn1_w[:, :NCAT * OUT].reshape(W, NCAT, OUT, 2 * OUT)
    w1n = gcn1_w[:, NCAT * OUT:]
    nblk = N // BN
    h1s, dinv = pl.pallas_call(
        _dense1_body,
        grid=(W, nblk),
        in_specs=[
            pl.BlockSpec((NCAT, 1, BN, OUT), lambda w, i: (0, w, i, 0)),
            pl.BlockSpec((1, BN, NNUM), lambda w, i: (w, i, 0)),
            pl.BlockSpec((1, NC, BN, 1), lambda w, i: (w, 0, i, 0)),
            pl.BlockSpec((1, NCAT, OUT, 2 * OUT), lambda w, i: (w, 0, 0, 0)),
            pl.BlockSpec((1, NNUM, 2 * OUT), lambda w, i: (w, 0, 0)),
        ],
        out_specs=[
            pl.BlockSpec((NC, 1, BN, OUT), lambda w, i: (0, w, i, 0)),
            pl.BlockSpec((1, BN, 1), lambda w, i: (w, i, 0)),
        ],
        out_shape=[
            jax.ShapeDtypeStruct((NC, W, N, OUT), jnp.float32),
            jax.ShapeDtypeStruct((W, N, 1), jnp.float32),
        ],
    )(cat, xnum, degp.reshape(W, NC, NPAD, 1), w1c, w1n)

    acc1 = _conv1(h1s.reshape(NC * W * N, OUT), s1, d1, zeros_hbm)

    b1 = gcn1_b.reshape(W, NC, OUT)
    w2 = gcn2_w.reshape(W, NC, OUT, OUT)
    h2 = pl.pallas_call(
        _dense2_body,
        grid=(W, nblk),
        in_specs=[
            pl.BlockSpec((1, NC, BN, OUT), lambda w, i: (w, 0, i, 0)),
            pl.BlockSpec((NC, 1, BN, OUT), lambda w, i: (0, w, i, 0)),
            pl.BlockSpec((1, BN, 1), lambda w, i: (w, i, 0)),
            pl.BlockSpec((1, NC, OUT), lambda w, i: (w, 0, 0)),
            pl.BlockSpec((1, NC, OUT, OUT), lambda w, i: (w, 0, 0, 0)),
        ],
        out_specs=pl.BlockSpec((1, BN, OUT), lambda w, i: (w, i, 0)),
        out_shape=jax.ShapeDtypeStruct((W, N, OUT), jnp.float32),
    )(acc1, h1s, dinv, b1, w2)

    acc2 = _conv2(h2.reshape(W * N, OUT), s2, d2, zeros_hbm)

    nblk_g = N // BG
    out = pl.pallas_call(
        _gru_body,
        grid=(nblk_g,),
        in_specs=[
            pl.BlockSpec((W, NC, BG, OUT), lambda i: (0, 0, i, 0)),
            pl.BlockSpec((W, BG, OUT), lambda i: (0, i, 0)),
            pl.BlockSpec((W, BG, 1), lambda i: (0, i, 0)),
            pl.BlockSpec((W, OUT), lambda i: (0, 0)),
            pl.BlockSpec((OUT, 3 * OUT), lambda i: (0, 0)),
            pl.BlockSpec((OUT, 3 * OUT), lambda i: (0, 0)),
            pl.BlockSpec((1, 3 * OUT), lambda i: (0, 0)),
            pl.BlockSpec((1, 3 * OUT), lambda i: (0, 0)),
        ],
        out_specs=pl.BlockSpec((BG, OUT), lambda i: (i, 0)),
        out_shape=jax.ShapeDtypeStruct((N, OUT), jnp.float32),
    )(acc2, h2, dinv, gcn2_b, w_ih.T, w_hh.T,
      b_ih.reshape(1, -1), b_hh.reshape(1, -1))
    return out
